# baseline (speedup 1.0000x reference)
import jax
import jax.numpy as jnp
import numpy as np
from jax.experimental import pallas as pl
from jax.experimental.pallas import tpu as pltpu

_N_TOK = 25088
_HW = 3136
_B = 8
_CIN = 128
_D = 64
_K = 512


def _vq_body(z_ref, pw_ref, pb_ref, cb_ref, idx_ref, q_ref, eloss_ref,
             counts_ref):
    z = z_ref[0]
    pw = pw_ref[...]
    cb = cb_ref[...]
    zf = (jnp.dot(pw, z, preferred_element_type=jnp.float32)
          + pb_ref[...].reshape(_D, 1))
    cb_sq = jnp.sum(cb * cb, axis=1)
    flat_sq = jnp.sum(zf * zf, axis=0, keepdims=True)
    mm = jnp.dot(cb, zf, preferred_element_type=jnp.float32)
    dist = (flat_sq + cb_sq[:, None]) - 2.0 * mm
    idx = jnp.argmin(dist, axis=0).astype(jnp.int32)
    idx_ref[0, 0, :] = idx
    enc = (jax.lax.broadcasted_iota(jnp.int32, dist.shape, 0)
           == idx[None, :]).astype(jnp.float32)
    q = jnp.dot(cb.T, enc, preferred_element_type=jnp.float32)
    q_ref[0] = q
    d = q - zf
    eloss_ref[...] = jnp.sum(d * d).reshape(1, 1, 1)
    counts_ref[0, 0, :] = jnp.sum(enc, axis=1)


def _vq_quantize(z, pre_w, pre_b, codebook):
    idx, q, eloss, counts = pl.pallas_call(
        _vq_body,
        grid=(_B,),
        compiler_params=pltpu.CompilerParams(
            dimension_semantics=("parallel",)),
        in_specs=[
            pl.BlockSpec((1, _CIN, _HW), lambda i: (i, 0, 0)),
            pl.BlockSpec((_D, _CIN), lambda i: (0, 0)),
            pl.BlockSpec((_D,), lambda i: (0,)),
            pl.BlockSpec((_K, _D), lambda i: (0, 0)),
        ],
        out_specs=[
            pl.BlockSpec((1, 1, _HW), lambda i: (i, 0, 0)),
            pl.BlockSpec((1, _D, _HW), lambda i: (i, 0, 0)),
            pl.BlockSpec((1, 1, 1), lambda i: (i, 0, 0)),
            pl.BlockSpec((1, 1, _K), lambda i: (i, 0, 0)),
        ],
        out_shape=[
            jax.ShapeDtypeStruct((_B, 1, _HW), jnp.int32),
            jax.ShapeDtypeStruct((_B, _D, _HW), jnp.float32),
            jax.ShapeDtypeStruct((_B, 1, 1), jnp.float32),
            jax.ShapeDtypeStruct((_B, 1, _K), jnp.float32),
        ],
    )(z, pre_w, pre_b, codebook)
    return (idx.reshape(_N_TOK), q, jnp.sum(eloss),
            jnp.sum(counts, axis=(0, 1)))


def _shift(x, s):
    if s == 0:
        return x
    c, l = x.shape
    z = jnp.zeros((c, abs(s)), x.dtype)
    if s > 0:
        return jnp.concatenate([x[:, s:], z], axis=1)
    return jnp.concatenate([z, x[:, :l + s]], axis=1)


def _masks():
    col = jax.lax.broadcasted_iota(jnp.int32, (1, _HW), 1) % 56
    return (col != 0).astype(jnp.float32), (col != 55).astype(jnp.float32)


def _shift_m(x, dy, dx, m0, m55):
    xs = _shift(x, 56 * dy + dx)
    if dx == 1:
        return xs * m55
    if dx == -1:
        return xs * m0
    return xs


def _conv3(x, w, m0, m55):
    rows = {dy: _shift(x, 56 * dy) for dy in (-1, 0, 1)}
    out = None
    for dx in (-1, 0, 1):
        acc = None
        for dy in (-1, 0, 1):
            t = jnp.dot(w[(dy + 1) * 3 + dx + 1], rows[dy],
                        preferred_element_type=jnp.float32)
            acc = t if acc is None else acc + t
        if dx:
            acc = _shift(acc, dx) * (m55 if dx == 1 else m0)
        out = acc if out is None else out + acc
    return out

_PHASE = {0: ((0, 1, -1), (2, 0, 0)), 1: ((1, 0, 0), (3, 1, 0)),
          2: ((0, 0, 0), (2, 1, 0)), 3: ((1, 1, 0), (3, 0, 1))}
_ROWVARS = ((0, 0), (1, -1), (1, 0), (0, 1))


def _dec_body(q_ref, dw1_ref, db1_ref, r1w1_ref, r1w2_ref, r2w1_ref,
              r2w2_ref, wt1_ref, bt1_ref, wt2_ref, out_ref):
    m0, m55 = _masks()
    q = q_ref[0]
    d = _conv3(q, dw1_ref[...], m0, m55) + db1_ref[...].reshape(-1, 1)
    for w1_ref, w2_ref in ((r1w1_ref, r1w2_ref), (r2w1_ref, r2w2_ref)):
        h = jax.nn.relu(d)
        h = jax.nn.relu(_conv3(h, w1_ref[...], m0, m55))
        d = d + jnp.dot(w2_ref[...], h, preferred_element_type=jnp.float32)
    d = jax.nn.relu(d)

    wt1 = wt1_ref[...]
    bt1 = bt1_ref[...].reshape(-1, 1)
    even = ((0, -1), (2, 0))
    odd = ((1, 0), (3, 1))
    drows = {dy: _shift(d, 56 * dy) for dy in (-1, 0, 1)}
    b = {}
    for dp, rows in ((0, even), (1, odd)):
        for ac in range(4):
            acc = None
            for (ar, dy) in rows:
                t = jnp.dot(wt1[ar * 4 + ac], drows[dy],
                            preferred_element_type=jnp.float32)
                acc = t if acc is None else acc + t
            b[(ac, dp)] = acc
    y = {}
    for dp in (0, 1):
        for dq, cols in ((0, even), (1, odd)):
            acc = None
            for (ac, dx) in cols:
                t = b[(ac, dp)]
                if dx:
                    t = _shift(t, dx) * (m55 if dx == 1 else m0)
                acc = t if acc is None else acc + t
            y[(dp, dq)] = jax.nn.relu(acc + bt1)

    yrows = {}
    for (rc, rdy) in _ROWVARS:
        for cc in (0, 1):
            yrows[(rc, rdy, cc)] = _shift(y[(rc, cc)], 56 * rdy)
    out = None
    for vj, (cc, cdx) in enumerate(_ROWVARS):
        acc = None
        for vi, (rc, rdy) in enumerate(_ROWVARS):
            t = jnp.dot(wt2_ref[vi * 4 + vj], yrows[(rc, rdy, cc)],
                        preferred_element_type=jnp.float32)
            acc = t if acc is None else acc + t
        if cdx:
            acc = _shift(acc, cdx) * (m55 if cdx == 1 else m0)
        out = acc if out is None else out + acc
    out_ref[0] = out


def _asm_body(in_ref, pz_ref, out_ref):
    pz = pz_ref[...]
    vs = []
    for pi in range(4):
        parts = [in_ref[0, 12 * pi + 3 * rho:12 * pi + 3 * rho + 3]
                 .reshape(168, 56) for rho in range(4)]
        v = jnp.dot(jnp.concatenate(parts, axis=1), pz,
                    preferred_element_type=jnp.float32)
        vs.append(v.reshape(3, 56, 224))
    for k in range(7):
        lo = 128 * k
        parts = []
        g = lo
        while g < lo + 128:
            pi = g // 224
            j0 = g % 224
            j1 = min(224, j0 + (lo + 128 - g))
            parts.append(vs[pi][:, :, j0:j1])
            g += j1 - j0
        wk = parts[0] if len(parts) == 1 else jnp.concatenate(parts, axis=-1)
        out_ref[pl.ds(0, 1), :, pl.Slice(k, 56, 7), :] = wk[None]


def _perm224():
    p = np.zeros((224, 224), np.float32)
    for rho in range(4):
        for u in range(56):
            p[56 * rho + u, 4 * u + rho] = 1.0
    return jnp.asarray(p)


def _assemble(cls4):
    out = pl.pallas_call(
        _asm_body,
        grid=(_B,),
        compiler_params=pltpu.CompilerParams(
            dimension_semantics=("parallel",)),
        in_specs=[pl.BlockSpec((1, 48, 56, 56), lambda i: (i, 0, 0, 0)),
                  pl.BlockSpec((224, 224), lambda i: (0, 0))],
        out_specs=pl.BlockSpec((1, 3, 392, 128), lambda i: (i, 0, 0, 0)),
        out_shape=jax.ShapeDtypeStruct((_B, 3, 392, 128), jnp.float32),
    )(cls4, _perm224())
    return out.reshape(_B, 3, 224, 224)


def _build_wt2(dt2_w):
    w2 = jnp.flip(dt2_w, axis=(2, 3)).transpose(1, 0, 2, 3)
    zeros = jnp.zeros((3, 64), jnp.float32)
    mats = []
    for (rc, rdy) in _ROWVARS:
        for (cc, cdx) in _ROWVARS:
            blocks = []
            for pi in range(4):
                ar = next((a for (a, c, s) in _PHASE[pi]
                           if (c, s) == (rc, rdy)), None)
                for rho in range(4):
                    ac = next((a for (a, c, s) in _PHASE[rho]
                               if (c, s) == (cc, cdx)), None)
                    blocks.append(w2[:, :, ar, ac]
                                  if ar is not None and ac is not None
                                  else zeros)
            mats.append(jnp.concatenate(blocks, axis=0))
    return jnp.stack(mats)


def _decode(q, dec_w1, dec_b1, dr1_w1, dr1_w2, dr2_w1, dr2_w2,
            dt1_w, dt1_b, dt2_w, dt2_b):
    w1t = dec_w1.transpose(2, 3, 0, 1).reshape(9, _CIN, _D)
    r1w1t = dr1_w1.transpose(2, 3, 0, 1).reshape(9, 32, _CIN)
    r2w1t = dr2_w1.transpose(2, 3, 0, 1).reshape(9, 32, _CIN)
    wt1 = (jnp.flip(dt1_w, axis=(2, 3)).transpose(1, 0, 2, 3)
           .transpose(2, 3, 0, 1).reshape(16, _D, _CIN))
    wt2 = _build_wt2(dt2_w)
    out_cls = pl.pallas_call(
        _dec_body,
        grid=(_B,),
        compiler_params=pltpu.CompilerParams(
            dimension_semantics=("parallel",)),
        in_specs=[
            pl.BlockSpec((1, _D, _HW), lambda i: (i, 0, 0)),
            pl.BlockSpec((9, _CIN, _D), lambda i: (0, 0, 0)),
            pl.BlockSpec((_CIN,), lambda i: (0,)),
            pl.BlockSpec((9, 32, _CIN), lambda i: (0, 0, 0)),
            pl.BlockSpec((_CIN, 32), lambda i: (0, 0)),
            pl.BlockSpec((9, 32, _CIN), lambda i: (0, 0, 0)),
            pl.BlockSpec((_CIN, 32), lambda i: (0, 0)),
            pl.BlockSpec((16, _D, _CIN), lambda i: (0, 0, 0)),
            pl.BlockSpec((_D,), lambda i: (0,)),
            pl.BlockSpec((16, 48, _D), lambda i: (0, 0, 0)),
        ],
        out_specs=pl.BlockSpec((1, 48, _HW), lambda i: (i, 0, 0)),
        out_shape=jax.ShapeDtypeStruct((_B, 48, _HW), jnp.float32),
    )(q, w1t, dec_b1, r1w1t, dr1_w2.reshape(_CIN, 32), r2w1t,
      dr2_w2.reshape(_CIN, 32), wt1, dt1_b, wt2)
    x = _assemble(out_cls.reshape(_B, 48, 56, 56))
    return x + dt2_b[None, :, None, None]


def _conv2d(x, w, b=None, stride=1, padding=0):
    out = jax.lax.conv_general_dilated(
        x, w, (stride, stride), [(padding, padding), (padding, padding)],
        dimension_numbers=('NCHW', 'OIHW', 'NCHW'))
    if b is not None:
        out = out + b[None, :, None, None]
    return out


def _conv_transpose2d(x, w, b, stride=2, padding=1):
    kh = w.shape[2]
    w2 = jnp.flip(w, axis=(2, 3)).transpose(1, 0, 2, 3)
    pad = kh - 1 - padding
    out = jax.lax.conv_general_dilated(
        x, w2, (1, 1), [(pad, pad), (pad, pad)], lhs_dilation=(stride, stride),
        dimension_numbers=('NCHW', 'OIHW', 'NCHW'))
    return out + b[None, :, None, None]


def _res_stack(x, p):
    for (w1, w2) in p:
        h = jax.nn.relu(x)
        h = _conv2d(h, w1, None, 1, 1)
        h = jax.nn.relu(h)
        h = _conv2d(h, w2, None, 1, 0)
        x = x + h
    return jax.nn.relu(x)


def kernel(x, enc_w1, enc_b1, enc_w2, enc_b2, enc_w3, enc_b3, er1_w1, er1_w2,
           er2_w1, er2_w2, pre_w, pre_b, codebook, dec_w1, dec_b1, dr1_w1,
           dr1_w2, dr2_w1, dr2_w2, dt1_w, dt1_b, dt2_w, dt2_b):
    z = jax.nn.relu(_conv2d(x, enc_w1, enc_b1, 2, 1))
    z = jax.nn.relu(_conv2d(z, enc_w2, enc_b2, 2, 1))
    z = _conv2d(z, enc_w3, enc_b3, 1, 1)
    z = _res_stack(z, [(er1_w1, er1_w2), (er2_w1, er2_w2)])

    idx, q, eloss_sum, counts = _vq_quantize(
        z.reshape(_B, _CIN, _HW), pre_w.reshape(_D, _CIN), pre_b, codebook)

    loss = 0.25 * (eloss_sum / (_N_TOK * _D))
    avg = counts / _N_TOK
    perp = jnp.exp(-jnp.sum(avg * jnp.log(avg + 1e-10)))
    x_recon = _decode(q, dec_w1, dec_b1, dr1_w1, dr1_w2, dr2_w1, dr2_w2,
                      dt1_w, dt1_b, dt2_w, dt2_b)
    return loss, x_recon, perp, idx[:, None]


# speedup vs baseline: 1.0123x; 1.0123x over previous
import jax
import jax.numpy as jnp
import numpy as np
from jax.experimental import pallas as pl

_N_TOK = 25088
_HW = 3136
_B = 8
_CIN = 128
_D = 64
_K = 512


def _vq_body(z_ref, pw_ref, pb_ref, cb_ref, idx_ref, q_ref, eloss_ref,
             counts_ref):
    z = z_ref[0]
    pw = pw_ref[...]
    cb = cb_ref[...]
    zf = (jnp.dot(pw, z, preferred_element_type=jnp.float32)
          + pb_ref[...].reshape(_D, 1))
    cb_sq = jnp.sum(cb * cb, axis=1)
    flat_sq = jnp.sum(zf * zf, axis=0, keepdims=True)
    mm = jnp.dot(cb, zf, preferred_element_type=jnp.float32)
    dist = (flat_sq + cb_sq[:, None]) - 2.0 * mm
    idx = jnp.argmin(dist, axis=0).astype(jnp.int32)
    idx_ref[0, 0, :] = idx
    enc = (jax.lax.broadcasted_iota(jnp.int32, dist.shape, 0)
           == idx[None, :]).astype(jnp.float32)
    q = jnp.dot(cb.T, enc, preferred_element_type=jnp.float32)
    q_ref[0] = q
    d = q - zf
    eloss_ref[...] = jnp.sum(d * d).reshape(1, 1, 1)
    counts_ref[0, 0, :] = jnp.sum(enc, axis=1)


def _vq_quantize(z, pre_w, pre_b, codebook):
    idx, q, eloss, counts = pl.pallas_call(
        _vq_body,
        grid=(_B,),
        in_specs=[
            pl.BlockSpec((1, _CIN, _HW), lambda i: (i, 0, 0)),
            pl.BlockSpec((_D, _CIN), lambda i: (0, 0)),
            pl.BlockSpec((_D,), lambda i: (0,)),
            pl.BlockSpec((_K, _D), lambda i: (0, 0)),
        ],
        out_specs=[
            pl.BlockSpec((1, 1, _HW), lambda i: (i, 0, 0)),
            pl.BlockSpec((1, _D, _HW), lambda i: (i, 0, 0)),
            pl.BlockSpec((1, 1, 1), lambda i: (i, 0, 0)),
            pl.BlockSpec((1, 1, _K), lambda i: (i, 0, 0)),
        ],
        out_shape=[
            jax.ShapeDtypeStruct((_B, 1, _HW), jnp.int32),
            jax.ShapeDtypeStruct((_B, _D, _HW), jnp.float32),
            jax.ShapeDtypeStruct((_B, 1, 1), jnp.float32),
            jax.ShapeDtypeStruct((_B, 1, _K), jnp.float32),
        ],
    )(z, pre_w, pre_b, codebook)
    return (idx.reshape(_N_TOK), q, jnp.sum(eloss),
            jnp.sum(counts, axis=(0, 1)))


def _shift(x, s):
    if s == 0:
        return x
    c, l = x.shape
    z = jnp.zeros((c, abs(s)), x.dtype)
    if s > 0:
        return jnp.concatenate([x[:, s:], z], axis=1)
    return jnp.concatenate([z, x[:, :l + s]], axis=1)


def _masks():
    col = jax.lax.broadcasted_iota(jnp.int32, (1, _HW), 1) % 56
    return (col != 0).astype(jnp.float32), (col != 55).astype(jnp.float32)


def _shift_m(x, dy, dx, m0, m55):
    xs = _shift(x, 56 * dy + dx)
    if dx == 1:
        return xs * m55
    if dx == -1:
        return xs * m0
    return xs


def _conv3(x, w, m0, m55):
    rows = {dy: _shift(x, 56 * dy) for dy in (-1, 0, 1)}
    out = None
    for dx in (-1, 0, 1):
        acc = None
        for dy in (-1, 0, 1):
            t = jnp.dot(w[(dy + 1) * 3 + dx + 1], rows[dy],
                        preferred_element_type=jnp.float32)
            acc = t if acc is None else acc + t
        if dx:
            acc = _shift(acc, dx) * (m55 if dx == 1 else m0)
        out = acc if out is None else out + acc
    return out

_PHASE = {0: ((0, 1, -1), (2, 0, 0)), 1: ((1, 0, 0), (3, 1, 0)),
          2: ((0, 0, 0), (2, 1, 0)), 3: ((1, 1, 0), (3, 0, 1))}
_ROWVARS = ((0, 0), (1, -1), (1, 0), (0, 1))


def _dec_body(q_ref, dw1_ref, db1_ref, r1w1_ref, r1w2_ref, r2w1_ref,
              r2w2_ref, wt1_ref, bt1_ref, wt2_ref, out_ref):
    m0, m55 = _masks()
    q = q_ref[0].astype(jnp.bfloat16)
    d = _conv3(q, dw1_ref[...], m0, m55) + db1_ref[...].reshape(-1, 1)
    for w1_ref, w2_ref in ((r1w1_ref, r1w2_ref), (r2w1_ref, r2w2_ref)):
        h = jax.nn.relu(d)
        h = jax.nn.relu(_conv3(h.astype(jnp.bfloat16), w1_ref[...], m0, m55))
        d = d + jnp.dot(w2_ref[...], h.astype(jnp.bfloat16),
                        preferred_element_type=jnp.float32)
    d = jax.nn.relu(d).astype(jnp.bfloat16)

    wt1 = wt1_ref[...]
    bt1 = bt1_ref[...].reshape(-1, 1)
    even = ((0, -1), (2, 0))
    odd = ((1, 0), (3, 1))
    drows = {dy: _shift(d, 56 * dy) for dy in (-1, 0, 1)}
    b = {}
    for dp, rows in ((0, even), (1, odd)):
        for ac in range(4):
            acc = None
            for (ar, dy) in rows:
                t = jnp.dot(wt1[ar * 4 + ac], drows[dy],
                            preferred_element_type=jnp.float32)
                acc = t if acc is None else acc + t
            b[(ac, dp)] = acc
    y = {}
    for dp in (0, 1):
        for dq, cols in ((0, even), (1, odd)):
            acc = None
            for (ac, dx) in cols:
                t = b[(ac, dp)]
                if dx:
                    t = _shift(t, dx) * (m55 if dx == 1 else m0)
                acc = t if acc is None else acc + t
            y[(dp, dq)] = jax.nn.relu(acc + bt1).astype(jnp.bfloat16)

    yrows = {}
    for (rc, rdy) in _ROWVARS:
        for cc in (0, 1):
            yrows[(rc, rdy, cc)] = _shift(y[(rc, cc)], 56 * rdy)
    out = None
    for vj, (cc, cdx) in enumerate(_ROWVARS):
        acc = None
        for vi, (rc, rdy) in enumerate(_ROWVARS):
            t = jnp.dot(wt2_ref[vi * 4 + vj], yrows[(rc, rdy, cc)],
                        preferred_element_type=jnp.float32)
            acc = t if acc is None else acc + t
        if cdx:
            acc = _shift(acc, cdx) * (m55 if cdx == 1 else m0)
        out = acc if out is None else out + acc
    out_ref[0] = out


def _asm_body(in_ref, pz_ref, out_ref):
    pz = pz_ref[...]
    vs = []
    for pi in range(4):
        parts = [in_ref[0, 12 * pi + 3 * rho:12 * pi + 3 * rho + 3]
                 .reshape(168, 56) for rho in range(4)]
        v = jnp.dot(jnp.concatenate(parts, axis=1), pz,
                    preferred_element_type=jnp.float32)
        vs.append(v.reshape(3, 56, 224))
    for k in range(7):
        lo = 128 * k
        parts = []
        g = lo
        while g < lo + 128:
            pi = g // 224
            j0 = g % 224
            j1 = min(224, j0 + (lo + 128 - g))
            parts.append(vs[pi][:, :, j0:j1])
            g += j1 - j0
        wk = parts[0] if len(parts) == 1 else jnp.concatenate(parts, axis=-1)
        out_ref[pl.ds(0, 1), :, pl.Slice(k, 56, 7), :] = wk[None]


def _perm224():
    p = np.zeros((224, 224), np.float32)
    for rho in range(4):
        for u in range(56):
            p[56 * rho + u, 4 * u + rho] = 1.0
    return jnp.asarray(p)


def _assemble(cls4):
    out = pl.pallas_call(
        _asm_body,
        grid=(_B,),
        in_specs=[pl.BlockSpec((1, 48, 56, 56), lambda i: (i, 0, 0, 0)),
                  pl.BlockSpec((224, 224), lambda i: (0, 0))],
        out_specs=pl.BlockSpec((1, 3, 392, 128), lambda i: (i, 0, 0, 0)),
        out_shape=jax.ShapeDtypeStruct((_B, 3, 392, 128), jnp.float32),
    )(cls4, _perm224())
    return out.reshape(_B, 3, 224, 224)


def _build_wt2(dt2_w):
    w2 = jnp.flip(dt2_w, axis=(2, 3)).transpose(1, 0, 2, 3)
    zeros = jnp.zeros((3, 64), jnp.float32)
    mats = []
    for (rc, rdy) in _ROWVARS:
        for (cc, cdx) in _ROWVARS:
            blocks = []
            for pi in range(4):
                ar = next((a for (a, c, s) in _PHASE[pi]
                           if (c, s) == (rc, rdy)), None)
                for rho in range(4):
                    ac = next((a for (a, c, s) in _PHASE[rho]
                               if (c, s) == (cc, cdx)), None)
                    blocks.append(w2[:, :, ar, ac]
                                  if ar is not None and ac is not None
                                  else zeros)
            mats.append(jnp.concatenate(blocks, axis=0))
    return jnp.stack(mats)


def _decode(q, dec_w1, dec_b1, dr1_w1, dr1_w2, dr2_w1, dr2_w2,
            dt1_w, dt1_b, dt2_w, dt2_b):
    bf = jnp.bfloat16
    w1t = dec_w1.transpose(2, 3, 0, 1).reshape(9, _CIN, _D).astype(bf)
    r1w1t = dr1_w1.transpose(2, 3, 0, 1).reshape(9, 32, _CIN).astype(bf)
    r2w1t = dr2_w1.transpose(2, 3, 0, 1).reshape(9, 32, _CIN).astype(bf)
    wt1 = (jnp.flip(dt1_w, axis=(2, 3)).transpose(1, 0, 2, 3)
           .transpose(2, 3, 0, 1).reshape(16, _D, _CIN).astype(bf))
    wt2 = _build_wt2(dt2_w).astype(bf)
    out_cls = pl.pallas_call(
        _dec_body,
        grid=(_B,),
        in_specs=[
            pl.BlockSpec((1, _D, _HW), lambda i: (i, 0, 0)),
            pl.BlockSpec((9, _CIN, _D), lambda i: (0, 0, 0)),
            pl.BlockSpec((_CIN,), lambda i: (0,)),
            pl.BlockSpec((9, 32, _CIN), lambda i: (0, 0, 0)),
            pl.BlockSpec((_CIN, 32), lambda i: (0, 0)),
            pl.BlockSpec((9, 32, _CIN), lambda i: (0, 0, 0)),
            pl.BlockSpec((_CIN, 32), lambda i: (0, 0)),
            pl.BlockSpec((16, _D, _CIN), lambda i: (0, 0, 0)),
            pl.BlockSpec((_D,), lambda i: (0,)),
            pl.BlockSpec((16, 48, _D), lambda i: (0, 0, 0)),
        ],
        out_specs=pl.BlockSpec((1, 48, _HW), lambda i: (i, 0, 0)),
        out_shape=jax.ShapeDtypeStruct((_B, 48, _HW), jnp.float32),
    )(q, w1t, dec_b1, r1w1t, dr1_w2.reshape(_CIN, 32).astype(bf), r2w1t,
      dr2_w2.reshape(_CIN, 32).astype(bf), wt1, dt1_b, wt2)
    x = _assemble(out_cls.reshape(_B, 48, 56, 56))
    return x + dt2_b[None, :, None, None]


def _conv2d(x, w, b=None, stride=1, padding=0):
    out = jax.lax.conv_general_dilated(
        x, w, (stride, stride), [(padding, padding), (padding, padding)],
        dimension_numbers=('NCHW', 'OIHW', 'NCHW'))
    if b is not None:
        out = out + b[None, :, None, None]
    return out


def _conv_transpose2d(x, w, b, stride=2, padding=1):
    kh = w.shape[2]
    w2 = jnp.flip(w, axis=(2, 3)).transpose(1, 0, 2, 3)
    pad = kh - 1 - padding
    out = jax.lax.conv_general_dilated(
        x, w2, (1, 1), [(pad, pad), (pad, pad)], lhs_dilation=(stride, stride),
        dimension_numbers=('NCHW', 'OIHW', 'NCHW'))
    return out + b[None, :, None, None]


def _res_stack(x, p):
    for (w1, w2) in p:
        h = jax.nn.relu(x)
        h = _conv2d(h, w1, None, 1, 1)
        h = jax.nn.relu(h)
        h = _conv2d(h, w2, None, 1, 0)
        x = x + h
    return jax.nn.relu(x)


def kernel(x, enc_w1, enc_b1, enc_w2, enc_b2, enc_w3, enc_b3, er1_w1, er1_w2,
           er2_w1, er2_w2, pre_w, pre_b, codebook, dec_w1, dec_b1, dr1_w1,
           dr1_w2, dr2_w1, dr2_w2, dt1_w, dt1_b, dt2_w, dt2_b):
    z = jax.nn.relu(_conv2d(x, enc_w1, enc_b1, 2, 1))
    z = jax.nn.relu(_conv2d(z, enc_w2, enc_b2, 2, 1))
    z = _conv2d(z, enc_w3, enc_b3, 1, 1)
    z = _res_stack(z, [(er1_w1, er1_w2), (er2_w1, er2_w2)])

    idx, q, eloss_sum, counts = _vq_quantize(
        z.reshape(_B, _CIN, _HW), pre_w.reshape(_D, _CIN), pre_b, codebook)

    loss = 0.25 * (eloss_sum / (_N_TOK * _D))
    avg = counts / _N_TOK
    perp = jnp.exp(-jnp.sum(avg * jnp.log(avg + 1e-10)))
    x_recon = _decode(q, dec_w1, dec_b1, dr1_w1, dr1_w2, dr2_w1, dr2_w2,
                      dt1_w, dt1_b, dt2_w, dt2_b)
    return loss, x_recon, perp, idx[:, None]
